# baseline (speedup 1.0000x reference)
import functools

import jax
import jax.numpy as jnp
from jax.experimental import pallas as pl
from jax.experimental.pallas import tpu as pltpu

_N = 10000
_F = 512


def _proj_kernel(x_ref, w_ref, h_ref):
    h_ref[:] = jax.lax.dot_general(
        x_ref[:], w_ref[:], (((1,), (1,)), ((), ())),
        preferred_element_type=jnp.float32)


def _spmm_kernel(adj_ref, h_ref, b_ref, a_ref, out_ref):
    o = jnp.dot(adj_ref[:], h_ref[:], preferred_element_type=jnp.float32)
    o = o + b_ref[:]
    out_ref[:] = jnp.where(o >= 0, o, a_ref[0, 0] * o)


def kernel(x, adj, W, b, prelu_a):
    BM1 = 2000
    h = pl.pallas_call(
        _proj_kernel,
        grid=(_N // BM1,),
        in_specs=[
            pl.BlockSpec((BM1, _F), lambda i: (i, 0)),
            pl.BlockSpec((_F, _F), lambda i: (0, 0)),
        ],
        out_specs=pl.BlockSpec((BM1, _F), lambda i: (i, 0)),
        out_shape=jax.ShapeDtypeStruct((_N, _F), jnp.float32),
    )(x, W)

    BM = 200
    out = pl.pallas_call(
        _spmm_kernel,
        grid=(_N // BM,),
        in_specs=[
            pl.BlockSpec((BM, _N), lambda i: (i, 0)),
            pl.BlockSpec((_N, _F), lambda i: (0, 0)),
            pl.BlockSpec((1, _F), lambda i: (0, 0)),
            pl.BlockSpec((1, 1), lambda i: (0, 0)),
        ],
        out_specs=pl.BlockSpec((BM, _F), lambda i: (i, 0)),
        out_shape=jax.ShapeDtypeStruct((_N, _F), jnp.float32),
        compiler_params=pltpu.CompilerParams(
            dimension_semantics=("arbitrary",)),
    )(adj, h, b.reshape(1, _F), prelu_a.reshape(1, 1))
    return out


# speedup vs baseline: 1.0229x; 1.0229x over previous
import jax
import jax.numpy as jnp
from jax.experimental import pallas as pl
from jax.experimental.pallas import tpu as pltpu

_N = 10000
_F = 512


def _proj_kernel(x_ref, w_ref, h_ref):
    h = jax.lax.dot_general(
        x_ref[:], w_ref[:], (((1,), (1,)), ((), ())),
        preferred_element_type=jnp.float32)
    h_ref[:] = h.astype(jnp.bfloat16)


def _spmm_kernel(adj_ref, h_ref, b_ref, a_ref, out_ref):
    o = jnp.dot(adj_ref[:].astype(jnp.bfloat16), h_ref[:],
                preferred_element_type=jnp.float32)
    o = o + b_ref[:]
    out_ref[:] = jnp.where(o >= 0, o, a_ref[0, 0] * o)


def kernel(x, adj, W, b, prelu_a):
    BM1 = 2000
    h = pl.pallas_call(
        _proj_kernel,
        grid=(_N // BM1,),
        in_specs=[
            pl.BlockSpec((BM1, _F), lambda i: (i, 0)),
            pl.BlockSpec((_F, _F), lambda i: (0, 0)),
        ],
        out_specs=pl.BlockSpec((BM1, _F), lambda i: (i, 0)),
        out_shape=jax.ShapeDtypeStruct((_N, _F), jnp.bfloat16),
    )(x, W)

    BM = 200
    out = pl.pallas_call(
        _spmm_kernel,
        grid=(_N // BM,),
        in_specs=[
            pl.BlockSpec((BM, _N), lambda i: (i, 0)),
            pl.BlockSpec((_N, _F), lambda i: (0, 0)),
            pl.BlockSpec((1, _F), lambda i: (0, 0)),
            pl.BlockSpec((1, 1), lambda i: (0, 0)),
        ],
        out_specs=pl.BlockSpec((BM, _F), lambda i: (i, 0)),
        out_shape=jax.ShapeDtypeStruct((_N, _F), jnp.float32),
        compiler_params=pltpu.CompilerParams(
            dimension_semantics=("arbitrary",)),
    )(adj, h, b.reshape(1, _F), prelu_a.reshape(1, 1))
    return out


# speedup vs baseline: 1.0695x; 1.0456x over previous
import jax
import jax.numpy as jnp
from jax.experimental import pallas as pl
from jax.experimental.pallas import tpu as pltpu

_N = 10000
_F = 512
_BM = 200
_BX = 2000
_NX = _N // _BX


def _fused_kernel(x_ref, w_ref, adj_ref, b_ref, a_ref, out_ref, h_ref):
    i = pl.program_id(0)

    @pl.when(i < _NX)
    def _():
        h = jax.lax.dot_general(
            x_ref[:], w_ref[:], (((1,), (1,)), ((), ())),
            preferred_element_type=jnp.float32)
        h_ref[pl.ds(i * _BX, _BX), :] = h

    @pl.when(i >= _NX)
    def _():
        o = jnp.dot(adj_ref[:], h_ref[:], preferred_element_type=jnp.float32)
        o = o + b_ref[:]
        out_ref[:] = jnp.where(o >= 0, o, a_ref[0, 0] * o)


def kernel(x, adj, W, b, prelu_a):
    grid = _N // _BM + _NX
    out = pl.pallas_call(
        _fused_kernel,
        grid=(grid,),
        in_specs=[
            pl.BlockSpec((_BX, _F), lambda i: (jnp.minimum(i, _NX - 1), 0)),
            pl.BlockSpec((_F, _F), lambda i: (0, 0)),
            pl.BlockSpec((_BM, _N), lambda i: (jnp.maximum(i - _NX, 0), 0)),
            pl.BlockSpec((1, _F), lambda i: (0, 0)),
            pl.BlockSpec((1, 1), lambda i: (0, 0)),
        ],
        out_specs=pl.BlockSpec((_BM, _F), lambda i: (jnp.maximum(i - _NX, 0), 0)),
        out_shape=jax.ShapeDtypeStruct((_N, _F), jnp.float32),
        scratch_shapes=[pltpu.VMEM((_N, _F), jnp.float32)],
        compiler_params=pltpu.CompilerParams(
            dimension_semantics=("arbitrary",)),
    )(x, W, adj, b.reshape(1, _F), prelu_a.reshape(1, 1))
    return out


# speedup vs baseline: 1.1869x; 1.1098x over previous
import jax
import jax.numpy as jnp
from jax.experimental import pallas as pl
from jax.experimental.pallas import tpu as pltpu

_N = 10000
_F = 512
_BM = 400
_BX = 2000
_NX = _N // _BX


def _fused_kernel(x_ref, w_ref, adj_ref, b_ref, a_ref, out_ref, h_ref):
    i = pl.program_id(0)

    @pl.when(i < _NX)
    def _():
        h = jax.lax.dot_general(
            x_ref[:], w_ref[:], (((1,), (1,)), ((), ())),
            preferred_element_type=jnp.float32)
        h_ref[pl.ds(i * _BX, _BX), :] = h.astype(jnp.bfloat16)

    @pl.when(i >= _NX)
    def _():
        o = jax.lax.dot_general(
            adj_ref[:].astype(jnp.bfloat16), h_ref[:],
            (((1,), (0,)), ((), ())),
            preferred_element_type=jnp.float32)
        o = o + b_ref[:]
        out_ref[:] = jnp.where(o >= 0, o, a_ref[0, 0] * o)


def kernel(x, adj, W, b, prelu_a):
    grid = _N // _BM + _NX
    out = pl.pallas_call(
        _fused_kernel,
        grid=(grid,),
        in_specs=[
            pl.BlockSpec((_BX, _F), lambda i: (jnp.minimum(i, _NX - 1), 0)),
            pl.BlockSpec((_F, _F), lambda i: (0, 0)),
            pl.BlockSpec((_BM, _N), lambda i: (jnp.maximum(i - _NX, 0), 0)),
            pl.BlockSpec((1, _F), lambda i: (0, 0)),
            pl.BlockSpec((1, 1), lambda i: (0, 0)),
        ],
        out_specs=pl.BlockSpec((_BM, _F), lambda i: (jnp.maximum(i - _NX, 0), 0)),
        out_shape=jax.ShapeDtypeStruct((_N, _F), jnp.float32),
        scratch_shapes=[pltpu.VMEM((_N, _F), jnp.bfloat16)],
        compiler_params=pltpu.CompilerParams(
            dimension_semantics=("arbitrary",)),
    )(x, W, adj, b.reshape(1, _F), prelu_a.reshape(1, 1))
    return out
